# baseline (speedup 1.0000x reference)
import functools

import jax
import jax.numpy as jnp
from jax import lax
from jax.experimental import pallas as pl
from jax.experimental.pallas import tpu as pltpu
from jax.experimental.pallas import tpu_sc as plsc

B, H, W, C = 4, 224, 224, 192
P = 16384
N = B * P
HW = H * W
C2 = 2 * C
NC, NS, L = 2, 16, 16
NW = NC * NS
NPTS = N // NW
CP = 32
NCH = NPTS // CP
CJ = C // L

_mesh = plsc.VectorSubcoreMesh(core_axis_name="c", subcore_axis_name="s")


@functools.partial(
    pl.kernel,
    mesh=_mesh,
    out_type=jax.ShapeDtypeStruct((N * C,), jnp.float32),
    compiler_params=pltpu.CompilerParams(use_tc_tiling_on_sc=False,
                                         needs_layout_passes=False),
    scratch_types=[
        pltpu.VMEM((2 * NPTS,), jnp.float32),
        pltpu.VMEM((2 * CP + L,), jnp.int32),
        pltpu.VMEM((2 * CP + L,), jnp.int32),
        pltpu.VMEM((4 * CP + L,), jnp.float32),
        pltpu.VMEM((4 * CP + L,), jnp.float32),
        pltpu.VMEM((2 * CP * C2,), jnp.float32),
        pltpu.VMEM((2 * CP * C2,), jnp.float32),
        pltpu.VMEM((CP * C,), jnp.float32),
        pltpu.VMEM((CP * C,), jnp.float32),
        pltpu.SemaphoreType.DMA,
        pltpu.SemaphoreType.DMA,
        pltpu.SemaphoreType.DMA,
        pltpu.SemaphoreType.DMA,
    ],
)
def _sampler(gridf, feat, out, gv, idx_a, idx_b, w_a, w_b,
             gb_a, gb_b, ob_a, ob_b, sem_g, sem_a, sem_b, sem_o):
    wid = lax.axis_index("s") * NC + lax.axis_index("c")
    q0 = wid * NPTS
    boff = (q0 // P) * HW

    cpg = pltpu.make_async_copy(gridf.at[pl.ds(2 * q0, 2 * NPTS)], gv, sem_g)
    cpg.start()
    cpg.wait()

    def compute_idx(c, idx_ref, w_ref):
        for s in range(CP // L):
            lb = c * CP + s * L
            pos = 2 * (lb + lax.iota(jnp.int32, L))
            gx = plsc.load_gather(gv, [pos])
            gy = plsc.load_gather(gv, [pos + 1])
            fx = gx * float(W) - 0.5
            fy = gy * float(H) - 0.5
            x0 = (fx + 1.0).astype(jnp.int32) - 1
            y0 = (fy + 1.0).astype(jnp.int32) - 1
            wx1 = fx - x0.astype(jnp.float32)
            wy1 = fy - y0.astype(jnp.float32)
            wx0m = jnp.where((x0 >= 0) & (x0 <= W - 1), 1.0 - wx1, 0.0)
            wx1m = jnp.where((x0 >= -1) & (x0 <= W - 2), wx1, 0.0)
            wy0m = jnp.where((y0 >= 0) & (y0 <= H - 1), 1.0 - wy1, 0.0)
            wy1m = jnp.where((y0 >= -1) & (y0 <= H - 2), wy1, 0.0)
            xs = jnp.clip(x0, 0, W - 2)
            wa = (jnp.where(xs == x0, wx0m, 0.0)
                  + jnp.where(xs == x0 + 1, wx1m, 0.0))
            wb = (jnp.where(xs + 1 == x0, wx0m, 0.0)
                  + jnp.where(xs == x0, wx1m, 0.0))
            yb0 = boff + jnp.clip(y0, 0, H - 1) * W + xs
            yb1 = boff + jnp.clip(y0 + 1, 0, H - 1) * W + xs
            o = s * L
            idx_ref[pl.ds(o, L)] = yb0 * C
            idx_ref[pl.ds(CP + o, L)] = yb1 * C
            w_ref[pl.ds(o, L)] = wy0m * wa
            w_ref[pl.ds(CP + o, L)] = wy0m * wb
            w_ref[pl.ds(2 * CP + o, L)] = wy1m * wa
            w_ref[pl.ds(3 * CP + o, L)] = wy1m * wb

    def fire(idx_ref, gb_ref, sem):
        def body(q, carry):
            o = pl.multiple_of(idx_ref[pl.ds(q, L)][0], 8)
            d = pl.multiple_of(q * C2, 8)
            pltpu.make_async_copy(feat.at[pl.ds(o, C2)],
                                  gb_ref.at[pl.ds(d, C2)], sem).start()
            return carry
        lax.fori_loop(0, 2 * CP, body, 0, unroll=4)

    def wait_gather(gb_ref, sem):
        pltpu.make_async_copy(feat.at[pl.ds(0, 2 * CP * C2)], gb_ref,
                              sem).wait()

    def combine(gb_ref, w_ref, ob_ref):
        def point(p, carry):
            b0 = p * C2
            b1 = (CP + p) * C2
            w0a = w_ref[pl.ds(p, L)][0]
            w0b = w_ref[pl.ds(CP + p, L)][0]
            w1a = w_ref[pl.ds(2 * CP + p, L)][0]
            w1b = w_ref[pl.ds(3 * CP + p, L)][0]
            for j in range(CJ):
                jo = j * L
                acc = (gb_ref[pl.ds(b0 + jo, L)] * w0a
                       + gb_ref[pl.ds(b0 + C + jo, L)] * w0b
                       + gb_ref[pl.ds(b1 + jo, L)] * w1a
                       + gb_ref[pl.ds(b1 + C + jo, L)] * w1b)
                ob_ref[pl.ds(p * C + jo, L)] = acc
            return carry
        lax.fori_loop(0, CP, point, 0, unroll=2)

    def store_out(c, ob_ref):
        cp = pltpu.make_async_copy(ob_ref,
                                   out.at[pl.ds((q0 + c * CP) * C, CP * C)],
                                   sem_o)
        cp.start()
        cp.wait()

    compute_idx(0, idx_a, w_a)
    fire(idx_a, gb_a, sem_a)
    compute_idx(1, idx_b, w_b)
    fire(idx_b, gb_b, sem_b)

    def chunk_pair(c2, carry):
        ca = 2 * c2
        wait_gather(gb_a, sem_a)
        combine(gb_a, w_a, ob_a)
        store_out(ca, ob_a)

        @pl.when(c2 < NCH // 2 - 1)
        def _():
            compute_idx(ca + 2, idx_a, w_a)
            fire(idx_a, gb_a, sem_a)

        wait_gather(gb_b, sem_b)
        combine(gb_b, w_b, ob_b)
        store_out(ca + 1, ob_b)

        @pl.when(c2 < NCH // 2 - 1)
        def _():
            compute_idx(ca + 3, idx_b, w_b)
            fire(idx_b, gb_b, sem_b)

        return carry

    lax.fori_loop(0, NCH // 2, chunk_pair, 0)


def kernel(features, grid):
    feat1 = features.reshape(B * H * W * C)
    grid1 = grid.reshape(2 * N)
    out = _sampler(grid1, feat1)
    return out.reshape(B, P, C)


# speedup vs baseline: 1.2098x; 1.2098x over previous
import functools

import jax
import jax.numpy as jnp
from jax import lax
from jax.experimental import pallas as pl
from jax.experimental.pallas import tpu as pltpu
from jax.experimental.pallas import tpu_sc as plsc

B, H, W, C = 4, 224, 224, 192
P = 16384
N = B * P
CPAD = 256
NC, NS, L = 2, 16, 16
NW = NC * NS
NPTS = N // NW
CP = 32
NROW = 4 * CP
NCH = NPTS // CP
CJ = C // L

_mesh = plsc.VectorSubcoreMesh(core_axis_name="c", subcore_axis_name="s")


def _pad_body(x_ref, o_ref):
    o_ref[:, 0:C] = x_ref[0, 0]
    o_ref[:, C:CPAD] = jnp.zeros((W, CPAD - C), jnp.float32)


_pad = pl.pallas_call(
    _pad_body,
    out_shape=jax.ShapeDtypeStruct((B * H * W, CPAD), jnp.float32),
    grid=(B * H,),
    in_specs=[pl.BlockSpec((1, 1, W, C), lambda i: (i // H, i % H, 0, 0))],
    out_specs=pl.BlockSpec((W, CPAD), lambda i: (i, 0)),
)


@functools.partial(
    pl.kernel,
    mesh=_mesh,
    out_type=jax.ShapeDtypeStruct((B, P, C), jnp.float32),
    compiler_params=pltpu.CompilerParams(needs_layout_passes=False),
    scratch_types=[
        pltpu.VMEM((2 * NPTS,), jnp.float32),
        pltpu.VMEM((NROW,), jnp.int32),
        pltpu.VMEM((NROW,), jnp.int32),
        pltpu.VMEM((NROW + L,), jnp.float32),
        pltpu.VMEM((NROW + L,), jnp.float32),
        pltpu.VMEM((NROW, CPAD), jnp.float32),
        pltpu.VMEM((NROW, CPAD), jnp.float32),
        pltpu.VMEM((CP, C), jnp.float32),
        pltpu.VMEM((CP, C), jnp.float32),
        pltpu.SemaphoreType.DMA,
        pltpu.SemaphoreType.DMA,
        pltpu.SemaphoreType.DMA,
        pltpu.SemaphoreType.DMA,
    ],
)
def _sampler(gridf, feat, out, gv, idx_a, idx_b, w_a, w_b,
             gb_a, gb_b, ob_a, ob_b, sem_g, sem_a, sem_b, sem_o):
    wid = lax.axis_index("s") * NC + lax.axis_index("c")
    q0 = wid * NPTS
    bi = q0 // P
    p0 = q0 - bi * P

    cpg = pltpu.make_async_copy(gridf.at[pl.ds(2 * q0, 2 * NPTS)], gv, sem_g)
    cpg.start()
    cpg.wait()

    def compute_idx(c, idx_ref, w_ref):
        for s in range(CP // L):
            lb = c * CP + s * L
            pos = 2 * (lb + lax.iota(jnp.int32, L))
            gx = plsc.load_gather(gv, [pos])
            gy = plsc.load_gather(gv, [pos + 1])
            fx = gx * float(W) - 0.5
            fy = gy * float(H) - 0.5
            x0 = (fx + 1.0).astype(jnp.int32) - 1
            y0 = (fy + 1.0).astype(jnp.int32) - 1
            wx1 = fx - x0.astype(jnp.float32)
            wy1 = fy - y0.astype(jnp.float32)
            wx0m = jnp.where((x0 >= 0) & (x0 <= W - 1), 1.0 - wx1, 0.0)
            wx1m = jnp.where((x0 >= -1) & (x0 <= W - 2), wx1, 0.0)
            wy0m = jnp.where((y0 >= 0) & (y0 <= H - 1), 1.0 - wy1, 0.0)
            wy1m = jnp.where((y0 >= -1) & (y0 <= H - 2), wy1, 0.0)
            xc0 = jnp.clip(x0, 0, W - 1)
            xc1 = jnp.clip(x0 + 1, 0, W - 1)
            yb0 = (bi * H + jnp.clip(y0, 0, H - 1)) * W
            yb1 = (bi * H + jnp.clip(y0 + 1, 0, H - 1)) * W
            o = s * L
            idx_ref[pl.ds(0 * CP + o, L)] = yb0 + xc0
            idx_ref[pl.ds(1 * CP + o, L)] = yb0 + xc1
            idx_ref[pl.ds(2 * CP + o, L)] = yb1 + xc0
            idx_ref[pl.ds(3 * CP + o, L)] = yb1 + xc1
            w_ref[pl.ds(0 * CP + o, L)] = wy0m * wx0m
            w_ref[pl.ds(1 * CP + o, L)] = wy0m * wx1m
            w_ref[pl.ds(2 * CP + o, L)] = wy1m * wx0m
            w_ref[pl.ds(3 * CP + o, L)] = wy1m * wx1m

    def gather_a():
        return pltpu.make_async_copy(feat.at[idx_a], gb_a, sem_a)

    def gather_b():
        return pltpu.make_async_copy(feat.at[idx_b], gb_b, sem_b)

    def combine(gb_ref, w_ref, ob_ref):
        def point(p, carry):
            w00 = w_ref[pl.ds(p, L)][0]
            w01 = w_ref[pl.ds(CP + p, L)][0]
            w10 = w_ref[pl.ds(2 * CP + p, L)][0]
            w11 = w_ref[pl.ds(3 * CP + p, L)][0]
            for j in range(CJ):
                sl = pl.ds(j * L, L)
                acc = (gb_ref[p, sl] * w00 + gb_ref[CP + p, sl] * w01
                       + gb_ref[2 * CP + p, sl] * w10
                       + gb_ref[3 * CP + p, sl] * w11)
                ob_ref[p, sl] = acc
            return carry
        lax.fori_loop(0, CP, point, 0, unroll=2)

    def store_out(c, ob_ref):
        cp = pltpu.make_async_copy(ob_ref, out.at[bi, pl.ds(p0 + c * CP, CP)],
                                   sem_o)
        cp.start()
        cp.wait()

    compute_idx(0, idx_a, w_a)
    gather_a().start()
    compute_idx(1, idx_b, w_b)
    gather_b().start()

    def chunk_pair(c2, carry):
        ca = 2 * c2
        gather_a().wait()
        combine(gb_a, w_a, ob_a)
        store_out(ca, ob_a)

        @pl.when(c2 < NCH // 2 - 1)
        def _():
            compute_idx(ca + 2, idx_a, w_a)
            gather_a().start()

        gather_b().wait()
        combine(gb_b, w_b, ob_b)
        store_out(ca + 1, ob_b)

        @pl.when(c2 < NCH // 2 - 1)
        def _():
            compute_idx(ca + 3, idx_b, w_b)
            gather_b().start()

        return carry

    lax.fori_loop(0, NCH // 2, chunk_pair, 0)


def kernel(features, grid):
    feat2 = _pad(features)
    grid1 = grid.reshape(2 * N)
    return _sampler(grid1, feat2)


# speedup vs baseline: 1.9261x; 1.5921x over previous
import functools

import jax
import jax.numpy as jnp
from jax import lax
from jax.experimental import pallas as pl
from jax.experimental.pallas import tpu as pltpu
from jax.experimental.pallas import tpu_sc as plsc

B, H, W, C = 4, 224, 224, 192
P = 16384
N = B * P
CPAD = 256
NC, NS, L = 2, 16, 16
NW = NC * NS
NPTS = N // NW
CP = 32
NROW = 4 * CP
NCH = NPTS // CP
CJ = C // L

_mesh = plsc.VectorSubcoreMesh(core_axis_name="c", subcore_axis_name="s")


KH = 8


def _pad_body(x_ref, o_ref):
    z = jnp.zeros((W, CPAD - C), jnp.float32)
    for k in range(KH):
        o_ref[pl.ds(k * W, W), :] = jnp.concatenate([x_ref[0, k], z], axis=1)


_pad = pl.pallas_call(
    _pad_body,
    out_shape=jax.ShapeDtypeStruct((B * H * W, CPAD), jnp.float32),
    grid=(B, H // KH),
    in_specs=[pl.BlockSpec((1, KH, W, C), lambda b, h: (b, h, 0, 0))],
    out_specs=pl.BlockSpec((KH * W, CPAD),
                           lambda b, h: (b * (H // KH) + h, 0)),
    compiler_params=pltpu.CompilerParams(
        dimension_semantics=("arbitrary", "arbitrary")),
)


@functools.partial(
    pl.kernel,
    mesh=_mesh,
    out_type=jax.ShapeDtypeStruct((B, P, C), jnp.float32),
    compiler_params=pltpu.CompilerParams(needs_layout_passes=False),
    scratch_types=[
        pltpu.VMEM((2 * NPTS,), jnp.float32),
        pltpu.VMEM((NROW,), jnp.int32),
        pltpu.VMEM((NROW,), jnp.int32),
        pltpu.VMEM((NROW + L,), jnp.float32),
        pltpu.VMEM((NROW + L,), jnp.float32),
        pltpu.VMEM((NROW, CPAD), jnp.float32),
        pltpu.VMEM((NROW, CPAD), jnp.float32),
        pltpu.VMEM((CP, C), jnp.float32),
        pltpu.VMEM((CP, C), jnp.float32),
        pltpu.SemaphoreType.DMA,
        pltpu.SemaphoreType.DMA,
        pltpu.SemaphoreType.DMA,
        pltpu.SemaphoreType.DMA,
    ],
)
def _sampler(gridf, feat, out, gv, idx_a, idx_b, w_a, w_b,
             gb_a, gb_b, ob_a, ob_b, sem_g, sem_a, sem_b, sem_o):
    wid = lax.axis_index("s") * NC + lax.axis_index("c")
    q0 = wid * NPTS
    bi = q0 // P
    p0 = q0 - bi * P

    cpg = pltpu.make_async_copy(gridf.at[pl.ds(2 * q0, 2 * NPTS)], gv, sem_g)
    cpg.start()
    cpg.wait()

    def compute_idx(c, idx_ref, w_ref):
        for s in range(CP // L):
            lb = c * CP + s * L
            pos = 2 * (lb + lax.iota(jnp.int32, L))
            gx = plsc.load_gather(gv, [pos])
            gy = plsc.load_gather(gv, [pos + 1])
            fx = gx * float(W) - 0.5
            fy = gy * float(H) - 0.5
            x0 = (fx + 1.0).astype(jnp.int32) - 1
            y0 = (fy + 1.0).astype(jnp.int32) - 1
            wx1 = fx - x0.astype(jnp.float32)
            wy1 = fy - y0.astype(jnp.float32)
            wx0m = jnp.where((x0 >= 0) & (x0 <= W - 1), 1.0 - wx1, 0.0)
            wx1m = jnp.where((x0 >= -1) & (x0 <= W - 2), wx1, 0.0)
            wy0m = jnp.where((y0 >= 0) & (y0 <= H - 1), 1.0 - wy1, 0.0)
            wy1m = jnp.where((y0 >= -1) & (y0 <= H - 2), wy1, 0.0)
            xc0 = jnp.clip(x0, 0, W - 1)
            xc1 = jnp.clip(x0 + 1, 0, W - 1)
            yb0 = (bi * H + jnp.clip(y0, 0, H - 1)) * W
            yb1 = (bi * H + jnp.clip(y0 + 1, 0, H - 1)) * W
            o = s * L
            idx_ref[pl.ds(0 * CP + o, L)] = yb0 + xc0
            idx_ref[pl.ds(1 * CP + o, L)] = yb0 + xc1
            idx_ref[pl.ds(2 * CP + o, L)] = yb1 + xc0
            idx_ref[pl.ds(3 * CP + o, L)] = yb1 + xc1
            w_ref[pl.ds(0 * CP + o, L)] = wy0m * wx0m
            w_ref[pl.ds(1 * CP + o, L)] = wy0m * wx1m
            w_ref[pl.ds(2 * CP + o, L)] = wy1m * wx0m
            w_ref[pl.ds(3 * CP + o, L)] = wy1m * wx1m

    def gather_a():
        return pltpu.make_async_copy(feat.at[idx_a], gb_a, sem_a)

    def gather_b():
        return pltpu.make_async_copy(feat.at[idx_b], gb_b, sem_b)

    def combine(gb_ref, w_ref, ob_ref):
        def point(p, carry):
            w00 = w_ref[pl.ds(p, L)][0]
            w01 = w_ref[pl.ds(CP + p, L)][0]
            w10 = w_ref[pl.ds(2 * CP + p, L)][0]
            w11 = w_ref[pl.ds(3 * CP + p, L)][0]
            for j in range(CJ):
                sl = pl.ds(j * L, L)
                acc = (gb_ref[p, sl] * w00 + gb_ref[CP + p, sl] * w01
                       + gb_ref[2 * CP + p, sl] * w10
                       + gb_ref[3 * CP + p, sl] * w11)
                ob_ref[p, sl] = acc
            return carry
        lax.fori_loop(0, CP, point, 0, unroll=2)

    def store_out(c, ob_ref):
        cp = pltpu.make_async_copy(ob_ref, out.at[bi, pl.ds(p0 + c * CP, CP)],
                                   sem_o)
        cp.start()
        cp.wait()

    compute_idx(0, idx_a, w_a)
    gather_a().start()
    compute_idx(1, idx_b, w_b)
    gather_b().start()

    def chunk_pair(c2, carry):
        ca = 2 * c2
        gather_a().wait()
        combine(gb_a, w_a, ob_a)
        store_out(ca, ob_a)

        @pl.when(c2 < NCH // 2 - 1)
        def _():
            compute_idx(ca + 2, idx_a, w_a)
            gather_a().start()

        gather_b().wait()
        combine(gb_b, w_b, ob_b)
        store_out(ca + 1, ob_b)

        @pl.when(c2 < NCH // 2 - 1)
        def _():
            compute_idx(ca + 3, idx_b, w_b)
            gather_b().start()

        return carry

    lax.fori_loop(0, NCH // 2, chunk_pair, 0)


def kernel(features, grid):
    feat2 = _pad(features)
    grid1 = grid.reshape(2 * N)
    return _sampler(grid1, feat2)


# speedup vs baseline: 2.5100x; 1.3032x over previous
import functools

import jax
import jax.numpy as jnp
from jax import lax
from jax.experimental import pallas as pl
from jax.experimental.pallas import tpu as pltpu
from jax.experimental.pallas import tpu_sc as plsc

B, H, W, C = 4, 224, 224, 192
P = 16384
N = B * P
CPAD = 256
NC, NS, L = 2, 16, 16
NW = NC * NS
NPTS = N // NW
CP = 32
NROW = 4 * CP
NCH = NPTS // CP
CJ = C // L

_mesh = plsc.VectorSubcoreMesh(core_axis_name="c", subcore_axis_name="s")


KH = 8


def _pad_body(x_ref, o_ref):
    z = jnp.zeros((W, CPAD - C), jnp.float32)
    for k in range(KH):
        v = jnp.transpose(x_ref[0, k], (1, 0))
        o_ref[pl.ds(k * W, W), :] = jnp.concatenate([v, z], axis=1)


_pad = pl.pallas_call(
    _pad_body,
    out_shape=jax.ShapeDtypeStruct((B * H * W, CPAD), jnp.float32),
    grid=(B, H // KH),
    in_specs=[pl.BlockSpec((1, KH, C, W), lambda b, h: (b, h, 0, 0))],
    out_specs=pl.BlockSpec((KH * W, CPAD),
                           lambda b, h: (b * (H // KH) + h, 0)),
    compiler_params=pltpu.CompilerParams(
        dimension_semantics=("arbitrary", "arbitrary")),
)


@functools.partial(
    pl.kernel,
    mesh=_mesh,
    out_type=jax.ShapeDtypeStruct((B, P, C), jnp.float32),
    compiler_params=pltpu.CompilerParams(needs_layout_passes=False),
    scratch_types=[
        pltpu.VMEM((2 * NPTS,), jnp.float32),
        pltpu.VMEM((NROW,), jnp.int32),
        pltpu.VMEM((NROW,), jnp.int32),
        pltpu.VMEM((NROW + L,), jnp.float32),
        pltpu.VMEM((NROW + L,), jnp.float32),
        pltpu.VMEM((NROW, CPAD), jnp.float32),
        pltpu.VMEM((NROW, CPAD), jnp.float32),
        pltpu.VMEM((CP, C), jnp.float32),
        pltpu.VMEM((CP, C), jnp.float32),
        pltpu.SemaphoreType.DMA,
        pltpu.SemaphoreType.DMA,
        pltpu.SemaphoreType.DMA,
        pltpu.SemaphoreType.DMA,
    ],
)
def _sampler(gridf, feat, out, gv, idx_a, idx_b, w_a, w_b,
             gb_a, gb_b, ob_a, ob_b, sem_g, sem_a, sem_b, sem_o):
    wid = lax.axis_index("s") * NC + lax.axis_index("c")
    q0 = wid * NPTS
    bi = q0 // P
    p0 = q0 - bi * P

    cpg = pltpu.make_async_copy(gridf.at[pl.ds(2 * q0, 2 * NPTS)], gv, sem_g)
    cpg.start()
    cpg.wait()

    def compute_idx(c, idx_ref, w_ref):
        for s in range(CP // L):
            lb = c * CP + s * L
            pos = 2 * (lb + lax.iota(jnp.int32, L))
            gx = plsc.load_gather(gv, [pos])
            gy = plsc.load_gather(gv, [pos + 1])
            fx = gx * float(W) - 0.5
            fy = gy * float(H) - 0.5
            x0 = (fx + 1.0).astype(jnp.int32) - 1
            y0 = (fy + 1.0).astype(jnp.int32) - 1
            wx1 = fx - x0.astype(jnp.float32)
            wy1 = fy - y0.astype(jnp.float32)
            wx0m = jnp.where((x0 >= 0) & (x0 <= W - 1), 1.0 - wx1, 0.0)
            wx1m = jnp.where((x0 >= -1) & (x0 <= W - 2), wx1, 0.0)
            wy0m = jnp.where((y0 >= 0) & (y0 <= H - 1), 1.0 - wy1, 0.0)
            wy1m = jnp.where((y0 >= -1) & (y0 <= H - 2), wy1, 0.0)
            xc0 = jnp.clip(x0, 0, W - 1)
            xc1 = jnp.clip(x0 + 1, 0, W - 1)
            yb0 = (bi * H + jnp.clip(y0, 0, H - 1)) * W
            yb1 = (bi * H + jnp.clip(y0 + 1, 0, H - 1)) * W
            o = s * L
            idx_ref[pl.ds(0 * CP + o, L)] = yb0 + xc0
            idx_ref[pl.ds(1 * CP + o, L)] = yb0 + xc1
            idx_ref[pl.ds(2 * CP + o, L)] = yb1 + xc0
            idx_ref[pl.ds(3 * CP + o, L)] = yb1 + xc1
            w_ref[pl.ds(0 * CP + o, L)] = wy0m * wx0m
            w_ref[pl.ds(1 * CP + o, L)] = wy0m * wx1m
            w_ref[pl.ds(2 * CP + o, L)] = wy1m * wx0m
            w_ref[pl.ds(3 * CP + o, L)] = wy1m * wx1m

    def gather_a():
        return pltpu.make_async_copy(feat.at[idx_a], gb_a, sem_a)

    def gather_b():
        return pltpu.make_async_copy(feat.at[idx_b], gb_b, sem_b)

    def combine(gb_ref, w_ref, ob_ref):
        def point(p, carry):
            w00 = w_ref[pl.ds(p, L)][0]
            w01 = w_ref[pl.ds(CP + p, L)][0]
            w10 = w_ref[pl.ds(2 * CP + p, L)][0]
            w11 = w_ref[pl.ds(3 * CP + p, L)][0]
            for j in range(CJ):
                sl = pl.ds(j * L, L)
                acc = (gb_ref[p, sl] * w00 + gb_ref[CP + p, sl] * w01
                       + gb_ref[2 * CP + p, sl] * w10
                       + gb_ref[3 * CP + p, sl] * w11)
                ob_ref[p, sl] = acc
            return carry
        lax.fori_loop(0, CP, point, 0, unroll=2)

    def store_out(c, ob_ref):
        cp = pltpu.make_async_copy(ob_ref, out.at[bi, pl.ds(p0 + c * CP, CP)],
                                   sem_o)
        cp.start()
        cp.wait()

    compute_idx(0, idx_a, w_a)
    gather_a().start()
    compute_idx(1, idx_b, w_b)
    gather_b().start()

    def chunk_pair(c2, carry):
        ca = 2 * c2
        gather_a().wait()
        combine(gb_a, w_a, ob_a)
        store_out(ca, ob_a)

        @pl.when(c2 < NCH // 2 - 1)
        def _():
            compute_idx(ca + 2, idx_a, w_a)
            gather_a().start()

        gather_b().wait()
        combine(gb_b, w_b, ob_b)
        store_out(ca + 1, ob_b)

        @pl.when(c2 < NCH // 2 - 1)
        def _():
            compute_idx(ca + 3, idx_b, w_b)
            gather_b().start()

        return carry

    lax.fori_loop(0, NCH // 2, chunk_pair, 0)


def kernel(features, grid):
    ft = jnp.transpose(features, (0, 1, 3, 2))
    feat2 = _pad(ft)
    grid1 = grid.reshape(2 * N)
    return _sampler(grid1, feat2)


# speedup vs baseline: 2.7344x; 1.0894x over previous
import functools

import jax
import jax.numpy as jnp
from jax import lax
from jax.experimental import pallas as pl
from jax.experimental.pallas import tpu as pltpu
from jax.experimental.pallas import tpu_sc as plsc

B, H, W, C = 4, 224, 224, 192
P = 16384
N = B * P
CPAD = 256
NC, NS, L = 2, 16, 16
NW = NC * NS
NPTS = N // NW
CP = 32
NROW = 4 * CP
NCH = NPTS // CP
CJ = C // L

_mesh = plsc.VectorSubcoreMesh(core_axis_name="c", subcore_axis_name="s")


KH = 8


def _pad_body(x_ref, o_ref):
    z = jnp.zeros((W, CPAD - C), jnp.float32)
    for k in range(KH):
        v = jnp.transpose(x_ref[0, k], (1, 0))
        o_ref[pl.ds(k * W, W), :] = jnp.concatenate([v, z], axis=1)


_pad = pl.pallas_call(
    _pad_body,
    out_shape=jax.ShapeDtypeStruct((B * H * W, CPAD), jnp.float32),
    grid=(B, H // KH),
    in_specs=[pl.BlockSpec((1, KH, C, W), lambda b, h: (b, h, 0, 0))],
    out_specs=pl.BlockSpec((KH * W, CPAD),
                           lambda b, h: (b * (H // KH) + h, 0)),
    compiler_params=pltpu.CompilerParams(
        dimension_semantics=("arbitrary", "arbitrary")),
)


@functools.partial(
    pl.kernel,
    mesh=_mesh,
    out_type=jax.ShapeDtypeStruct((B, P, C), jnp.float32),
    compiler_params=pltpu.CompilerParams(needs_layout_passes=False),
    scratch_types=[
        pltpu.VMEM((NPTS,), jnp.float32),
        pltpu.VMEM((NPTS,), jnp.float32),
        pltpu.VMEM((NROW,), jnp.int32),
        pltpu.VMEM((NROW,), jnp.int32),
        pltpu.VMEM((NROW + L,), jnp.float32),
        pltpu.VMEM((NROW + L,), jnp.float32),
        pltpu.VMEM((NROW, CPAD), jnp.float32),
        pltpu.VMEM((NROW, CPAD), jnp.float32),
        pltpu.VMEM((CP, C), jnp.float32),
        pltpu.VMEM((CP, C), jnp.float32),
        pltpu.SemaphoreType.DMA,
        pltpu.SemaphoreType.DMA,
        pltpu.SemaphoreType.DMA,
        pltpu.SemaphoreType.DMA,
    ],
)
def _sampler(gridf, feat, out, gx_v, gy_v, idx_a, idx_b, w_a, w_b,
             gb_a, gb_b, ob_a, ob_b, sem_g, sem_a, sem_b, sem_o):
    wid = lax.axis_index("s") * NC + lax.axis_index("c")
    q0 = wid * NPTS
    bi = q0 // P
    p0 = q0 - bi * P

    cpx = pltpu.make_async_copy(gridf.at[bi, 0, pl.ds(p0, NPTS)], gx_v, sem_g)
    cpy = pltpu.make_async_copy(gridf.at[bi, 1, pl.ds(p0, NPTS)], gy_v, sem_g)
    cpx.start()
    cpy.start()
    cpx.wait()
    cpy.wait()

    def compute_idx(c, idx_ref, w_ref):
        for s in range(CP // L):
            lb = c * CP + s * L
            gx = gx_v[pl.ds(lb, L)]
            gy = gy_v[pl.ds(lb, L)]
            fx = gx * float(W) - 0.5
            fy = gy * float(H) - 0.5
            x0 = (fx + 1.0).astype(jnp.int32) - 1
            y0 = (fy + 1.0).astype(jnp.int32) - 1
            wx1 = fx - x0.astype(jnp.float32)
            wy1 = fy - y0.astype(jnp.float32)
            wx0m = jnp.where((x0 >= 0) & (x0 <= W - 1), 1.0 - wx1, 0.0)
            wx1m = jnp.where((x0 >= -1) & (x0 <= W - 2), wx1, 0.0)
            wy0m = jnp.where((y0 >= 0) & (y0 <= H - 1), 1.0 - wy1, 0.0)
            wy1m = jnp.where((y0 >= -1) & (y0 <= H - 2), wy1, 0.0)
            xc0 = jnp.clip(x0, 0, W - 1)
            xc1 = jnp.clip(x0 + 1, 0, W - 1)
            yb0 = (bi * H + jnp.clip(y0, 0, H - 1)) * W
            yb1 = (bi * H + jnp.clip(y0 + 1, 0, H - 1)) * W
            o = s * L
            idx_ref[pl.ds(0 * CP + o, L)] = yb0 + xc0
            idx_ref[pl.ds(1 * CP + o, L)] = yb0 + xc1
            idx_ref[pl.ds(2 * CP + o, L)] = yb1 + xc0
            idx_ref[pl.ds(3 * CP + o, L)] = yb1 + xc1
            w_ref[pl.ds(0 * CP + o, L)] = wy0m * wx0m
            w_ref[pl.ds(1 * CP + o, L)] = wy0m * wx1m
            w_ref[pl.ds(2 * CP + o, L)] = wy1m * wx0m
            w_ref[pl.ds(3 * CP + o, L)] = wy1m * wx1m

    def gather_a():
        return pltpu.make_async_copy(feat.at[idx_a], gb_a, sem_a)

    def gather_b():
        return pltpu.make_async_copy(feat.at[idx_b], gb_b, sem_b)

    def combine(gb_ref, w_ref, ob_ref):
        def point(p, carry):
            w00 = w_ref[pl.ds(p, L)][0]
            w01 = w_ref[pl.ds(CP + p, L)][0]
            w10 = w_ref[pl.ds(2 * CP + p, L)][0]
            w11 = w_ref[pl.ds(3 * CP + p, L)][0]
            for j in range(CJ):
                sl = pl.ds(j * L, L)
                acc = (gb_ref[p, sl] * w00 + gb_ref[CP + p, sl] * w01
                       + gb_ref[2 * CP + p, sl] * w10
                       + gb_ref[3 * CP + p, sl] * w11)
                ob_ref[p, sl] = acc
            return carry
        lax.fori_loop(0, CP, point, 0, unroll=2)

    def store_out(c, ob_ref):
        cp = pltpu.make_async_copy(ob_ref, out.at[bi, pl.ds(p0 + c * CP, CP)],
                                   sem_o)
        cp.start()
        cp.wait()

    compute_idx(0, idx_a, w_a)
    gather_a().start()
    compute_idx(1, idx_b, w_b)
    gather_b().start()

    def chunk_pair(c2, carry):
        ca = 2 * c2
        gather_a().wait()
        combine(gb_a, w_a, ob_a)
        store_out(ca, ob_a)

        @pl.when(c2 < NCH // 2 - 1)
        def _():
            compute_idx(ca + 2, idx_a, w_a)
            gather_a().start()

        gather_b().wait()
        combine(gb_b, w_b, ob_b)
        store_out(ca + 1, ob_b)

        @pl.when(c2 < NCH // 2 - 1)
        def _():
            compute_idx(ca + 3, idx_b, w_b)
            gather_b().start()

        return carry

    lax.fori_loop(0, NCH // 2, chunk_pair, 0)


def kernel(features, grid):
    ft = jnp.transpose(features, (0, 1, 3, 2))
    feat2 = _pad(ft)
    gxy = jnp.transpose(grid, (0, 2, 1))
    return _sampler(gxy, feat2)


# speedup vs baseline: 2.8357x; 1.0370x over previous
import functools

import jax
import jax.numpy as jnp
from jax import lax
from jax.experimental import pallas as pl
from jax.experimental.pallas import tpu as pltpu
from jax.experimental.pallas import tpu_sc as plsc

B, H, W, C = 4, 224, 224, 192
P = 16384
N = B * P
CPAD = 256
NC, NS, L = 2, 16, 16
NW = NC * NS
NPTS = N // NW
CP = 32
NROW = 4 * CP
NCH = NPTS // CP
CJ = C // L

_mesh = plsc.VectorSubcoreMesh(core_axis_name="c", subcore_axis_name="s")


KH = 8


def _pad_body(x_ref, o_ref):
    z = jnp.zeros((W, CPAD - C), jnp.float32)
    for k in range(KH):
        v = jnp.transpose(x_ref[0, k], (1, 0))
        o_ref[pl.ds(k * W, W), :] = jnp.concatenate([v, z], axis=1)


_pad = pl.pallas_call(
    _pad_body,
    out_shape=jax.ShapeDtypeStruct((B * H * W, CPAD), jnp.float32),
    grid=(B, H // KH),
    in_specs=[pl.BlockSpec((1, KH, C, W), lambda b, h: (b, h, 0, 0))],
    out_specs=pl.BlockSpec((KH * W, CPAD),
                           lambda b, h: (b * (H // KH) + h, 0)),
    compiler_params=pltpu.CompilerParams(
        dimension_semantics=("arbitrary", "arbitrary")),
)


@functools.partial(
    pl.kernel,
    mesh=_mesh,
    out_type=jax.ShapeDtypeStruct((B, P, C), jnp.float32),
    compiler_params=pltpu.CompilerParams(needs_layout_passes=False),
    scratch_types=[
        pltpu.VMEM((NPTS,), jnp.float32),
        pltpu.VMEM((NPTS,), jnp.float32),
        pltpu.VMEM((NROW,), jnp.int32),
        pltpu.VMEM((NROW,), jnp.int32),
        pltpu.VMEM((NROW + L,), jnp.float32),
        pltpu.VMEM((NROW + L,), jnp.float32),
        pltpu.VMEM((NROW, CPAD), jnp.float32),
        pltpu.VMEM((NROW, CPAD), jnp.float32),
        pltpu.VMEM((CP, C), jnp.float32),
        pltpu.VMEM((CP, C), jnp.float32),
        pltpu.SemaphoreType.DMA,
        pltpu.SemaphoreType.DMA,
        pltpu.SemaphoreType.DMA,
        pltpu.SemaphoreType.DMA,
        pltpu.SemaphoreType.DMA,
    ],
)
def _sampler(gridf, feat, out, gx_v, gy_v, idx_a, idx_b, w_a, w_b,
             gb_a, gb_b, ob_a, ob_b, sem_g, sem_a, sem_b, sem_oa, sem_ob):
    wid = lax.axis_index("s") * NC + lax.axis_index("c")
    q0 = wid * NPTS
    bi = q0 // P
    p0 = q0 - bi * P

    cpx = pltpu.make_async_copy(gridf.at[bi, 0, pl.ds(p0, NPTS)], gx_v, sem_g)
    cpy = pltpu.make_async_copy(gridf.at[bi, 1, pl.ds(p0, NPTS)], gy_v, sem_g)
    cpx.start()
    cpy.start()
    cpx.wait()
    cpy.wait()

    def compute_idx(c, idx_ref, w_ref):
        for s in range(CP // L):
            lb = c * CP + s * L
            gx = gx_v[pl.ds(lb, L)]
            gy = gy_v[pl.ds(lb, L)]
            fx = gx * float(W) - 0.5
            fy = gy * float(H) - 0.5
            x0 = (fx + 1.0).astype(jnp.int32) - 1
            y0 = (fy + 1.0).astype(jnp.int32) - 1
            wx1 = fx - x0.astype(jnp.float32)
            wy1 = fy - y0.astype(jnp.float32)
            wx0m = jnp.where((x0 >= 0) & (x0 <= W - 1), 1.0 - wx1, 0.0)
            wx1m = jnp.where((x0 >= -1) & (x0 <= W - 2), wx1, 0.0)
            wy0m = jnp.where((y0 >= 0) & (y0 <= H - 1), 1.0 - wy1, 0.0)
            wy1m = jnp.where((y0 >= -1) & (y0 <= H - 2), wy1, 0.0)
            xc0 = jnp.clip(x0, 0, W - 1)
            xc1 = jnp.clip(x0 + 1, 0, W - 1)
            yb0 = (bi * H + jnp.clip(y0, 0, H - 1)) * W
            yb1 = (bi * H + jnp.clip(y0 + 1, 0, H - 1)) * W
            o = s * L
            idx_ref[pl.ds(0 * CP + o, L)] = yb0 + xc0
            idx_ref[pl.ds(1 * CP + o, L)] = yb0 + xc1
            idx_ref[pl.ds(2 * CP + o, L)] = yb1 + xc0
            idx_ref[pl.ds(3 * CP + o, L)] = yb1 + xc1
            w_ref[pl.ds(0 * CP + o, L)] = wy0m * wx0m
            w_ref[pl.ds(1 * CP + o, L)] = wy0m * wx1m
            w_ref[pl.ds(2 * CP + o, L)] = wy1m * wx0m
            w_ref[pl.ds(3 * CP + o, L)] = wy1m * wx1m

    def gather_a():
        return pltpu.make_async_copy(feat.at[idx_a], gb_a, sem_a)

    def gather_b():
        return pltpu.make_async_copy(feat.at[idx_b], gb_b, sem_b)

    def combine(gb_ref, w_ref, ob_ref):
        def point(p, carry):
            w00 = w_ref[pl.ds(p, L)][0]
            w01 = w_ref[pl.ds(CP + p, L)][0]
            w10 = w_ref[pl.ds(2 * CP + p, L)][0]
            w11 = w_ref[pl.ds(3 * CP + p, L)][0]
            for j in range(CJ):
                sl = pl.ds(j * L, L)
                acc = (gb_ref[p, sl] * w00 + gb_ref[CP + p, sl] * w01
                       + gb_ref[2 * CP + p, sl] * w10
                       + gb_ref[3 * CP + p, sl] * w11)
                ob_ref[p, sl] = acc
            return carry
        lax.fori_loop(0, CP, point, 0, unroll=4)

    def store_out(c, ob_ref, sem):
        pltpu.make_async_copy(ob_ref, out.at[bi, pl.ds(p0 + c * CP, CP)],
                              sem).start()

    def wait_store(ob_ref, sem):
        pltpu.make_async_copy(ob_ref, out.at[bi, pl.ds(p0, CP)], sem).wait()

    compute_idx(0, idx_a, w_a)
    gather_a().start()
    compute_idx(1, idx_b, w_b)
    gather_b().start()

    def chunk_pair(c2, carry):
        ca = 2 * c2
        gather_a().wait()

        @pl.when(c2 > 0)
        def _():
            wait_store(ob_a, sem_oa)
        combine(gb_a, w_a, ob_a)
        store_out(ca, ob_a, sem_oa)

        @pl.when(c2 < NCH // 2 - 1)
        def _():
            compute_idx(ca + 2, idx_a, w_a)
            gather_a().start()

        gather_b().wait()

        @pl.when(c2 > 0)
        def _():
            wait_store(ob_b, sem_ob)
        combine(gb_b, w_b, ob_b)
        store_out(ca + 1, ob_b, sem_ob)

        @pl.when(c2 < NCH // 2 - 1)
        def _():
            compute_idx(ca + 3, idx_b, w_b)
            gather_b().start()

        return carry

    lax.fori_loop(0, NCH // 2, chunk_pair, 0)
    wait_store(ob_a, sem_oa)
    wait_store(ob_b, sem_ob)


def kernel(features, grid):
    ft = jnp.transpose(features, (0, 1, 3, 2))
    feat2 = _pad(ft)
    gxy = jnp.transpose(grid, (0, 2, 1))
    return _sampler(gxy, feat2)


# speedup vs baseline: 3.0471x; 1.0746x over previous
import functools

import jax
import jax.numpy as jnp
from jax import lax
from jax.experimental import pallas as pl
from jax.experimental.pallas import tpu as pltpu
from jax.experimental.pallas import tpu_sc as plsc

B, H, W, C = 4, 224, 224, 192
P = 16384
N = B * P
CPAD = 256
NC, NS, L = 2, 16, 16
NW = NC * NS
NPTS = N // NW
CP = 32
NROW = 4 * CP
NCH = NPTS // CP
CJ = C // L

_mesh = plsc.VectorSubcoreMesh(core_axis_name="c", subcore_axis_name="s")


KH = 16


def _pad_body(x_ref, o_ref):
    z = jnp.zeros((W, CPAD - C), jnp.float32)
    for k in range(KH):
        v = jnp.transpose(x_ref[0, k], (1, 0))
        o_ref[pl.ds(k * W, W), :] = jnp.concatenate([v, z], axis=1)


_pad = pl.pallas_call(
    _pad_body,
    out_shape=jax.ShapeDtypeStruct((B * H * W, CPAD), jnp.float32),
    grid=(B, H // KH),
    in_specs=[pl.BlockSpec((1, KH, C, W), lambda b, h: (b, h, 0, 0))],
    out_specs=pl.BlockSpec((KH * W, CPAD),
                           lambda b, h: (b * (H // KH) + h, 0)),
    compiler_params=pltpu.CompilerParams(
        dimension_semantics=("arbitrary", "arbitrary")),
)


@functools.partial(
    pl.kernel,
    mesh=_mesh,
    out_type=jax.ShapeDtypeStruct((B, P, C), jnp.float32),
    compiler_params=pltpu.CompilerParams(needs_layout_passes=False),
    scratch_types=[
        pltpu.VMEM((NPTS,), jnp.float32),
        pltpu.VMEM((NPTS,), jnp.float32),
        pltpu.VMEM((NROW,), jnp.int32),
        pltpu.VMEM((NROW,), jnp.int32),
        pltpu.VMEM((NROW + L,), jnp.float32),
        pltpu.VMEM((NROW + L,), jnp.float32),
        pltpu.VMEM((NROW, CPAD), jnp.float32),
        pltpu.VMEM((NROW, CPAD), jnp.float32),
        pltpu.VMEM((CP, C), jnp.float32),
        pltpu.VMEM((CP, C), jnp.float32),
        pltpu.SemaphoreType.DMA,
        pltpu.SemaphoreType.DMA,
        pltpu.SemaphoreType.DMA,
        pltpu.SemaphoreType.DMA,
        pltpu.SemaphoreType.DMA,
    ],
)
def _sampler(gridf, feat, out, gx_v, gy_v, idx_a, idx_b, w_a, w_b,
             gb_a, gb_b, ob_a, ob_b, sem_g, sem_a, sem_b, sem_oa, sem_ob):
    wid = lax.axis_index("s") * NC + lax.axis_index("c")
    q0 = wid * NPTS
    bi = q0 // P
    p0 = q0 - bi * P

    cpx = pltpu.make_async_copy(gridf.at[bi, 0, pl.ds(p0, NPTS)], gx_v, sem_g)
    cpy = pltpu.make_async_copy(gridf.at[bi, 1, pl.ds(p0, NPTS)], gy_v, sem_g)
    cpx.start()
    cpy.start()
    cpx.wait()
    cpy.wait()

    def compute_idx(c, idx_ref, w_ref):
        for s in range(CP // L):
            lb = c * CP + s * L
            gx = gx_v[pl.ds(lb, L)]
            gy = gy_v[pl.ds(lb, L)]
            fx = gx * float(W) - 0.5
            fy = gy * float(H) - 0.5
            x0 = (fx + 1.0).astype(jnp.int32) - 1
            y0 = (fy + 1.0).astype(jnp.int32) - 1
            wx1 = fx - x0.astype(jnp.float32)
            wy1 = fy - y0.astype(jnp.float32)
            wx0m = jnp.where((x0 >= 0) & (x0 <= W - 1), 1.0 - wx1, 0.0)
            wx1m = jnp.where((x0 >= -1) & (x0 <= W - 2), wx1, 0.0)
            wy0m = jnp.where((y0 >= 0) & (y0 <= H - 1), 1.0 - wy1, 0.0)
            wy1m = jnp.where((y0 >= -1) & (y0 <= H - 2), wy1, 0.0)
            xc0 = jnp.clip(x0, 0, W - 1)
            xc1 = jnp.clip(x0 + 1, 0, W - 1)
            yb0 = (bi * H + jnp.clip(y0, 0, H - 1)) * W
            yb1 = (bi * H + jnp.clip(y0 + 1, 0, H - 1)) * W
            o = s * L
            idx_ref[pl.ds(0 * CP + o, L)] = yb0 + xc0
            idx_ref[pl.ds(1 * CP + o, L)] = yb0 + xc1
            idx_ref[pl.ds(2 * CP + o, L)] = yb1 + xc0
            idx_ref[pl.ds(3 * CP + o, L)] = yb1 + xc1
            w_ref[pl.ds(0 * CP + o, L)] = wy0m * wx0m
            w_ref[pl.ds(1 * CP + o, L)] = wy0m * wx1m
            w_ref[pl.ds(2 * CP + o, L)] = wy1m * wx0m
            w_ref[pl.ds(3 * CP + o, L)] = wy1m * wx1m

    def gather_a():
        return pltpu.make_async_copy(feat.at[idx_a], gb_a, sem_a)

    def gather_b():
        return pltpu.make_async_copy(feat.at[idx_b], gb_b, sem_b)

    def combine(gb_ref, w_ref, ob_ref):
        def point(p, carry):
            w00 = w_ref[pl.ds(p, L)][0]
            w01 = w_ref[pl.ds(CP + p, L)][0]
            w10 = w_ref[pl.ds(2 * CP + p, L)][0]
            w11 = w_ref[pl.ds(3 * CP + p, L)][0]
            for j in range(CJ):
                sl = pl.ds(j * L, L)
                acc = (gb_ref[p, sl] * w00 + gb_ref[CP + p, sl] * w01
                       + gb_ref[2 * CP + p, sl] * w10
                       + gb_ref[3 * CP + p, sl] * w11)
                ob_ref[p, sl] = acc
            return carry
        lax.fori_loop(0, CP, point, 0, unroll=4)

    def store_out(c, ob_ref, sem):
        pltpu.make_async_copy(ob_ref, out.at[bi, pl.ds(p0 + c * CP, CP)],
                              sem).start()

    def wait_store(ob_ref, sem):
        pltpu.make_async_copy(ob_ref, out.at[bi, pl.ds(p0, CP)], sem).wait()

    compute_idx(0, idx_a, w_a)
    gather_a().start()
    compute_idx(1, idx_b, w_b)
    gather_b().start()

    def chunk_pair(c2, carry):
        ca = 2 * c2
        gather_a().wait()

        @pl.when(c2 > 0)
        def _():
            wait_store(ob_a, sem_oa)
        combine(gb_a, w_a, ob_a)
        store_out(ca, ob_a, sem_oa)

        @pl.when(c2 < NCH // 2 - 1)
        def _():
            compute_idx(ca + 2, idx_a, w_a)
            gather_a().start()

        gather_b().wait()

        @pl.when(c2 > 0)
        def _():
            wait_store(ob_b, sem_ob)
        combine(gb_b, w_b, ob_b)
        store_out(ca + 1, ob_b, sem_ob)

        @pl.when(c2 < NCH // 2 - 1)
        def _():
            compute_idx(ca + 3, idx_b, w_b)
            gather_b().start()

        return carry

    lax.fori_loop(0, NCH // 2, chunk_pair, 0)
    wait_store(ob_a, sem_oa)
    wait_store(ob_b, sem_ob)


def kernel(features, grid):
    ft = jnp.transpose(features, (0, 1, 3, 2))
    feat2 = _pad(ft)
    gxy = jnp.transpose(grid, (0, 2, 1))
    return _sampler(gxy, feat2)


# speedup vs baseline: 3.0896x; 1.0139x over previous
import functools

import jax
import jax.numpy as jnp
from jax import lax
from jax.experimental import pallas as pl
from jax.experimental.pallas import tpu as pltpu
from jax.experimental.pallas import tpu_sc as plsc

B, H, W, C = 4, 224, 224, 192
P = 16384
N = B * P
CPAD = 256
NC, NS, L = 2, 16, 16
NW = NC * NS
NPTS = N // NW
CP = 32
NROW = 4 * CP
NCH = NPTS // CP
CJ = C // L

_mesh = plsc.VectorSubcoreMesh(core_axis_name="c", subcore_axis_name="s")


KH = 28


def _pad_body(x_ref, o_ref):
    z = jnp.zeros((W, CPAD - C), jnp.float32)
    for k in range(KH):
        v = jnp.transpose(x_ref[0, k], (1, 0))
        o_ref[pl.ds(k * W, W), :] = jnp.concatenate([v, z], axis=1)


_pad = pl.pallas_call(
    _pad_body,
    out_shape=jax.ShapeDtypeStruct((B * H * W, CPAD), jnp.float32),
    grid=(B, H // KH),
    in_specs=[pl.BlockSpec((1, KH, C, W), lambda b, h: (b, h, 0, 0))],
    out_specs=pl.BlockSpec((KH * W, CPAD),
                           lambda b, h: (b * (H // KH) + h, 0)),
    compiler_params=pltpu.CompilerParams(
        dimension_semantics=("arbitrary", "arbitrary")),
)


@functools.partial(
    pl.kernel,
    mesh=_mesh,
    out_type=jax.ShapeDtypeStruct((B, P, C), jnp.float32),
    compiler_params=pltpu.CompilerParams(needs_layout_passes=False),
    scratch_types=[
        pltpu.VMEM((NPTS,), jnp.float32),
        pltpu.VMEM((NPTS,), jnp.float32),
        pltpu.VMEM((NROW,), jnp.int32),
        pltpu.VMEM((NROW,), jnp.int32),
        pltpu.VMEM((NROW + L,), jnp.float32),
        pltpu.VMEM((NROW + L,), jnp.float32),
        pltpu.VMEM((NROW, CPAD), jnp.float32),
        pltpu.VMEM((NROW, CPAD), jnp.float32),
        pltpu.VMEM((CP, C), jnp.float32),
        pltpu.VMEM((CP, C), jnp.float32),
        pltpu.SemaphoreType.DMA,
        pltpu.SemaphoreType.DMA,
        pltpu.SemaphoreType.DMA,
        pltpu.SemaphoreType.DMA,
        pltpu.SemaphoreType.DMA,
    ],
)
def _sampler(gridf, feat, out, gx_v, gy_v, idx_a, idx_b, w_a, w_b,
             gb_a, gb_b, ob_a, ob_b, sem_g, sem_a, sem_b, sem_oa, sem_ob):
    wid = lax.axis_index("s") * NC + lax.axis_index("c")
    q0 = wid * NPTS
    bi = q0 // P
    p0 = q0 - bi * P

    cpx = pltpu.make_async_copy(gridf.at[bi, 0, pl.ds(p0, NPTS)], gx_v, sem_g)
    cpy = pltpu.make_async_copy(gridf.at[bi, 1, pl.ds(p0, NPTS)], gy_v, sem_g)
    cpx.start()
    cpy.start()
    cpx.wait()
    cpy.wait()

    def compute_idx(c, idx_ref, w_ref):
        for s in range(CP // L):
            lb = c * CP + s * L
            gx = gx_v[pl.ds(lb, L)]
            gy = gy_v[pl.ds(lb, L)]
            fx = gx * float(W) - 0.5
            fy = gy * float(H) - 0.5
            x0 = (fx + 1.0).astype(jnp.int32) - 1
            y0 = (fy + 1.0).astype(jnp.int32) - 1
            wx1 = fx - x0.astype(jnp.float32)
            wy1 = fy - y0.astype(jnp.float32)
            wx0m = jnp.where((x0 >= 0) & (x0 <= W - 1), 1.0 - wx1, 0.0)
            wx1m = jnp.where((x0 >= -1) & (x0 <= W - 2), wx1, 0.0)
            wy0m = jnp.where((y0 >= 0) & (y0 <= H - 1), 1.0 - wy1, 0.0)
            wy1m = jnp.where((y0 >= -1) & (y0 <= H - 2), wy1, 0.0)
            xc0 = jnp.clip(x0, 0, W - 1)
            xc1 = jnp.clip(x0 + 1, 0, W - 1)
            yb0 = (bi * H + jnp.clip(y0, 0, H - 1)) * W
            yb1 = (bi * H + jnp.clip(y0 + 1, 0, H - 1)) * W
            o = s * L
            idx_ref[pl.ds(0 * CP + o, L)] = yb0 + xc0
            idx_ref[pl.ds(1 * CP + o, L)] = yb0 + xc1
            idx_ref[pl.ds(2 * CP + o, L)] = yb1 + xc0
            idx_ref[pl.ds(3 * CP + o, L)] = yb1 + xc1
            w_ref[pl.ds(0 * CP + o, L)] = wy0m * wx0m
            w_ref[pl.ds(1 * CP + o, L)] = wy0m * wx1m
            w_ref[pl.ds(2 * CP + o, L)] = wy1m * wx0m
            w_ref[pl.ds(3 * CP + o, L)] = wy1m * wx1m

    def gather_a():
        return pltpu.make_async_copy(feat.at[idx_a], gb_a, sem_a)

    def gather_b():
        return pltpu.make_async_copy(feat.at[idx_b], gb_b, sem_b)

    def combine(gb_ref, w_ref, ob_ref):
        def point(p, carry):
            w00 = w_ref[pl.ds(p, L)][0]
            w01 = w_ref[pl.ds(CP + p, L)][0]
            w10 = w_ref[pl.ds(2 * CP + p, L)][0]
            w11 = w_ref[pl.ds(3 * CP + p, L)][0]
            for j in range(CJ):
                sl = pl.ds(j * L, L)
                acc = (gb_ref[p, sl] * w00 + gb_ref[CP + p, sl] * w01
                       + gb_ref[2 * CP + p, sl] * w10
                       + gb_ref[3 * CP + p, sl] * w11)
                ob_ref[p, sl] = acc
            return carry
        lax.fori_loop(0, CP, point, 0, unroll=4)

    def store_out(c, ob_ref, sem):
        pltpu.make_async_copy(ob_ref, out.at[bi, pl.ds(p0 + c * CP, CP)],
                              sem).start()

    def wait_store(ob_ref, sem):
        pltpu.make_async_copy(ob_ref, out.at[bi, pl.ds(p0, CP)], sem).wait()

    compute_idx(0, idx_a, w_a)
    gather_a().start()
    compute_idx(1, idx_b, w_b)
    gather_b().start()

    def chunk_pair(c2, carry):
        ca = 2 * c2
        gather_a().wait()

        @pl.when(c2 > 0)
        def _():
            wait_store(ob_a, sem_oa)
        combine(gb_a, w_a, ob_a)
        store_out(ca, ob_a, sem_oa)

        @pl.when(c2 < NCH // 2 - 1)
        def _():
            compute_idx(ca + 2, idx_a, w_a)
            gather_a().start()

        gather_b().wait()

        @pl.when(c2 > 0)
        def _():
            wait_store(ob_b, sem_ob)
        combine(gb_b, w_b, ob_b)
        store_out(ca + 1, ob_b, sem_ob)

        @pl.when(c2 < NCH // 2 - 1)
        def _():
            compute_idx(ca + 3, idx_b, w_b)
            gather_b().start()

        return carry

    lax.fori_loop(0, NCH // 2, chunk_pair, 0)
    wait_store(ob_a, sem_oa)
    wait_store(ob_b, sem_ob)


def kernel(features, grid):
    ft = jnp.transpose(features, (0, 1, 3, 2))
    feat2 = _pad(ft)
    gxy = jnp.transpose(grid, (0, 2, 1))
    return _sampler(gxy, feat2)
